# baseline (speedup 1.0000x reference)
import jax
import jax.numpy as jnp
from jax.experimental import pallas as pl
from jax.experimental.pallas import tpu as pltpu

_IN = 16
_OUT = 7
_CHUNK = 8192
_NBUF = 4


def _make_pipeline_kernel(nchunks):
    def _pipeline_kernel(x_hbm, w_ref, y_hbm, xbuf, ybuf, insem, outsem):
        def in_copy(chunk, slot):
            return pltpu.make_async_copy(
                x_hbm.at[pl.ds(chunk * _CHUNK, _CHUNK), :],
                xbuf.at[slot],
                insem.at[slot],
            )

        def out_copy(chunk, slot):
            return pltpu.make_async_copy(
                ybuf.at[slot],
                y_hbm.at[pl.ds(chunk * _CHUNK, _CHUNK), :],
                outsem.at[slot],
            )

        for b in range(_NBUF):
            in_copy(b, b).start()

        w = w_ref[...]

        def body(i, carry):
            slot = jax.lax.rem(i, _NBUF)
            in_copy(i, slot).wait()

            @pl.when(i >= _NBUF)
            def _():
                out_copy(i - _NBUF, slot).wait()

            ybuf[slot] = jnp.dot(xbuf[slot], w, preferred_element_type=jnp.float32)
            out_copy(i, slot).start()

            @pl.when(i + _NBUF < nchunks)
            def _():
                in_copy(i + _NBUF, slot).start()

            return carry

        jax.lax.fori_loop(0, nchunks, body, 0)

        for k in range(_NBUF):
            c = nchunks - _NBUF + k
            out_copy(c, c % _NBUF).wait()

    return _pipeline_kernel


def kernel(x, w):
    n, in_feats = x.shape
    assert in_feats == _IN and w.shape == (_IN, _OUT)
    assert n % _CHUNK == 0
    nchunks = n // _CHUNK

    return pl.pallas_call(
        _make_pipeline_kernel(nchunks),
        out_shape=jax.ShapeDtypeStruct((n, _OUT), x.dtype),
        in_specs=[
            pl.BlockSpec(memory_space=pltpu.MemorySpace.HBM),
            pl.BlockSpec(memory_space=pltpu.MemorySpace.VMEM),
        ],
        out_specs=pl.BlockSpec(memory_space=pltpu.MemorySpace.HBM),
        scratch_shapes=[
            pltpu.VMEM((_NBUF, _CHUNK, _IN), jnp.float32),
            pltpu.VMEM((_NBUF, _CHUNK, _OUT), jnp.float32),
            pltpu.SemaphoreType.DMA((_NBUF,)),
            pltpu.SemaphoreType.DMA((_NBUF,)),
        ],
        cost_estimate=pl.CostEstimate(
            flops=2 * n * _IN * _OUT,
            transcendentals=0,
            bytes_accessed=(n * (_IN + _OUT) + _IN * _OUT) * x.dtype.itemsize,
        ),
    )(x, w)


# speedup vs baseline: 1.0003x; 1.0003x over previous
import jax
import jax.numpy as jnp
from jax.experimental import pallas as pl
from jax.experimental.pallas import tpu as pltpu

_IN = 16
_OUT = 7
_CHUNK = 8192
_NBUF = 4


def _make_pipeline_kernel(nchunks):
    nquart = nchunks // 4

    def _pipeline_kernel(x_hbm, w_ref, y_hbm, xbuf, ybuf, insem, outsem):
        def perm(i):
            return jax.lax.rem(i, 4) * nquart + jax.lax.div(i, 4)

        def in_copy(chunk, slot):
            return pltpu.make_async_copy(
                x_hbm.at[pl.ds(chunk * _CHUNK, _CHUNK), :],
                xbuf.at[slot],
                insem.at[slot],
            )

        def out_copy(chunk, slot):
            return pltpu.make_async_copy(
                ybuf.at[slot],
                y_hbm.at[pl.ds(chunk * _CHUNK, _CHUNK), :],
                outsem.at[slot],
            )

        for b in range(_NBUF):
            in_copy(b * nquart, b).start()

        w = w_ref[...]

        def body(i, carry):
            slot = jax.lax.rem(i, _NBUF)
            in_copy(perm(i), slot).wait()

            @pl.when(i >= _NBUF)
            def _():
                out_copy(perm(i - _NBUF), slot).wait()

            ybuf[slot] = jnp.dot(xbuf[slot], w, preferred_element_type=jnp.float32)
            out_copy(perm(i), slot).start()

            @pl.when(i + _NBUF < nchunks)
            def _():
                in_copy(perm(i + _NBUF), slot).start()

            return carry

        jax.lax.fori_loop(0, nchunks, body, 0)

        for k in range(_NBUF):
            i = nchunks - _NBUF + k
            c = (i % 4) * nquart + i // 4
            out_copy(c, i % _NBUF).wait()

    return _pipeline_kernel


def kernel(x, w):
    n, in_feats = x.shape
    assert in_feats == _IN and w.shape == (_IN, _OUT)
    assert n % (4 * _CHUNK) == 0
    nchunks = n // _CHUNK

    return pl.pallas_call(
        _make_pipeline_kernel(nchunks),
        out_shape=jax.ShapeDtypeStruct((n, _OUT), x.dtype),
        in_specs=[
            pl.BlockSpec(memory_space=pltpu.MemorySpace.HBM),
            pl.BlockSpec(memory_space=pltpu.MemorySpace.VMEM),
        ],
        out_specs=pl.BlockSpec(memory_space=pltpu.MemorySpace.HBM),
        scratch_shapes=[
            pltpu.VMEM((_NBUF, _CHUNK, _IN), jnp.float32),
            pltpu.VMEM((_NBUF, _CHUNK, _OUT), jnp.float32),
            pltpu.SemaphoreType.DMA((_NBUF,)),
            pltpu.SemaphoreType.DMA((_NBUF,)),
        ],
        cost_estimate=pl.CostEstimate(
            flops=2 * n * _IN * _OUT,
            transcendentals=0,
            bytes_accessed=(n * (_IN + _OUT) + _IN * _OUT) * x.dtype.itemsize,
        ),
    )(x, w)


# speedup vs baseline: 1.1506x; 1.1502x over previous
import jax
import jax.numpy as jnp
from jax.experimental import pallas as pl
from jax.experimental.pallas import tpu as pltpu

_IN = 16
_OUT = 7
_TILE = 8192


def _dual_stream_kernel(lo_ref, hi_ref, w_ref, o_ref):
    w = w_ref[...]
    o_ref[0] = jnp.dot(lo_ref[...], w, preferred_element_type=jnp.float32)
    o_ref[1] = jnp.dot(hi_ref[...], w, preferred_element_type=jnp.float32)


def kernel(x, w):
    n, in_feats = x.shape
    assert in_feats == _IN and w.shape == (_IN, _OUT)
    assert n % (2 * _TILE) == 0
    half = n // 2
    steps = half // _TILE
    hi_base = steps

    y2 = pl.pallas_call(
        _dual_stream_kernel,
        out_shape=jax.ShapeDtypeStruct((2, half, _OUT), x.dtype),
        grid=(steps,),
        in_specs=[
            pl.BlockSpec((_TILE, _IN), lambda i: (i, 0)),
            pl.BlockSpec((_TILE, _IN), lambda i: (i + hi_base, 0)),
            pl.BlockSpec((_IN, _OUT), lambda i: (0, 0)),
        ],
        out_specs=pl.BlockSpec((2, _TILE, _OUT), lambda i: (0, i, 0)),
        compiler_params=pltpu.CompilerParams(
            dimension_semantics=("parallel",),
        ),
        cost_estimate=pl.CostEstimate(
            flops=2 * n * _IN * _OUT,
            transcendentals=0,
            bytes_accessed=(n * (_IN + _OUT) + _IN * _OUT) * x.dtype.itemsize,
        ),
    )(x, x, w)

    return y2.reshape(n, _OUT)
